# speedup vs baseline: 1.3560x; 1.3560x over previous
import jax
import jax.numpy as jnp
from jax.experimental import pallas as pl
from jax.experimental.pallas import tpu as pltpu


def _body(mod_ref, x_ref, cb_ref, out_ref):
    x = x_ref[0]
    cb = cb_ref[0]
    n, k = out_ref.shape[1], out_ref.shape[2]
    cross = jax.lax.dot_general(
        x, cb, (((1,), (1,)), ((), ())),
        preferred_element_type=jnp.float32)
    z2 = jnp.sum(x * x, axis=1, keepdims=True)
    c2 = jnp.sum(cb * cb, axis=1)[None, :]
    dists = z2 - 2.0 * cross + c2
    minval = jnp.min(dists, axis=1, keepdims=True)
    iota = jax.lax.broadcasted_iota(jnp.int32, (n, k), 1)
    code = jnp.min(jnp.where(dists == minval, iota, k), axis=1, keepdims=True)
    out_ref[0] = (iota == code).astype(jnp.int32)


def kernel(input, mod, codebooks):
    b, n, d = input.shape
    m, k, _ = codebooks.shape
    grid_spec = pltpu.PrefetchScalarGridSpec(
        num_scalar_prefetch=1,
        grid=(b,),
        in_specs=[
            pl.BlockSpec((1, n, d), lambda i, mod_ref: (i, 0, 0)),
            pl.BlockSpec((1, k, d), lambda i, mod_ref: (mod_ref[i], 0, 0)),
        ],
        out_specs=pl.BlockSpec((1, n, k), lambda i, mod_ref: (i, 0, 0)),
    )
    out = pl.pallas_call(
        _body,
        grid_spec=grid_spec,
        out_shape=jax.ShapeDtypeStruct((b, n, k), jnp.int32),
    )(mod, input, codebooks)
    return out.reshape(b, n * k)


# speedup vs baseline: 2.2516x; 1.6605x over previous
import jax
import jax.numpy as jnp
from jax.experimental import pallas as pl
from jax.experimental.pallas import tpu as pltpu

_CN = 36
_NSTEP = 16


def _codes_body(mod_ref, x_ref, cb_ref, out_ref):
    x = x_ref[0]
    cb = cb_ref[0]
    k = cb.shape[0]
    cross_t = jax.lax.dot_general(
        cb, x, (((1,), (1,)), ((), ())),
        preferred_element_type=jnp.float32)
    c2 = jnp.sum(cb * cb, axis=1, keepdims=True)
    dists = c2 - 2.0 * cross_t
    minval = jnp.min(dists, axis=0, keepdims=True)
    iota_k = jax.lax.broadcasted_iota(jnp.int32, dists.shape, 0)
    code = jnp.min(jnp.where(dists == minval, iota_k, k),
                   axis=0, keepdims=True)
    for s in range(_NSTEP):
        out_ref[0, 0:1, s * 128:s * 128 + _CN] = code[:, s * _CN:(s + 1) * _CN]


def _onehot_body(codes_ref, out_ref):
    bdim = out_ref.shape[0]
    k = out_ref.shape[1] // _CN
    codes_blk = codes_ref[:, 0, :]
    iota = jax.lax.broadcasted_iota(jnp.int32, (bdim, k), 1)
    for j in range(_CN):
        code = codes_blk[:, j:j + 1]
        out_ref[:, j * k:(j + 1) * k] = (iota == code).astype(jnp.int32)


def kernel(input, mod, codebooks):
    b, n, d = input.shape
    m, k, _ = codebooks.shape

    codes_spec = pltpu.PrefetchScalarGridSpec(
        num_scalar_prefetch=1,
        grid=(b,),
        in_specs=[
            pl.BlockSpec((1, n, d), lambda i, mod_ref: (i, 0, 0)),
            pl.BlockSpec((1, k, d), lambda i, mod_ref: (mod_ref[i], 0, 0)),
        ],
        out_specs=pl.BlockSpec((1, 1, _NSTEP * 128), lambda i, mod_ref: (i, 0, 0)),
    )
    codes = pl.pallas_call(
        _codes_body,
        grid_spec=codes_spec,
        out_shape=jax.ShapeDtypeStruct((b, 1, _NSTEP * 128), jnp.int32),
    )(mod, input, codebooks)

    chunk = _CN * k
    out = pl.pallas_call(
        _onehot_body,
        grid=(_NSTEP,),
        in_specs=[pl.BlockSpec((b, 1, 128), lambda s: (0, 0, s))],
        out_specs=pl.BlockSpec((b, chunk), lambda s: (0, s)),
        out_shape=jax.ShapeDtypeStruct((b, n * k), jnp.int32),
    )(codes)
    return out


# speedup vs baseline: 2.4171x; 1.0735x over previous
import jax
import jax.numpy as jnp
from jax.experimental import pallas as pl
from jax.experimental.pallas import tpu as pltpu

_CN = 72
_NSTEP = 8


def _codes_body(mod_ref, x_ref, cb_ref, out_ref):
    x = x_ref[0]
    cb = cb_ref[0]
    k = cb.shape[0]
    xm2 = x * -2.0
    cross_t = jax.lax.dot_general(
        cb, xm2, (((1,), (1,)), ((), ())),
        preferred_element_type=jnp.float32)
    c2 = jax.lax.dot_general(
        cb * cb, jnp.ones((1, cb.shape[1]), jnp.float32),
        (((1,), (1,)), ((), ())),
        preferred_element_type=jnp.float32)
    dists = c2 + cross_t
    minval = jnp.min(dists, axis=0, keepdims=True)
    iota_k = jax.lax.broadcasted_iota(jnp.int32, dists.shape, 0)
    code = jnp.min(jnp.where(dists == minval, iota_k, k),
                   axis=0, keepdims=True)
    for s in range(_NSTEP):
        out_ref[0, 0:1, s * 128:s * 128 + _CN] = code[:, s * _CN:(s + 1) * _CN]


def _onehot_body(codes_ref, out_ref):
    bdim = out_ref.shape[0]
    k = out_ref.shape[1] // _CN
    codes_blk = codes_ref[:, 0, :]
    iota = jax.lax.broadcasted_iota(jnp.int32, (bdim, k), 1)
    for j in range(_CN):
        code = codes_blk[:, j:j + 1]
        out_ref[:, j * k:(j + 1) * k] = (iota == code).astype(jnp.int32)


def kernel(input, mod, codebooks):
    b, n, d = input.shape
    m, k, _ = codebooks.shape

    codes_spec = pltpu.PrefetchScalarGridSpec(
        num_scalar_prefetch=1,
        grid=(b,),
        in_specs=[
            pl.BlockSpec((1, n, d), lambda i, mod_ref: (i, 0, 0)),
            pl.BlockSpec((1, k, d), lambda i, mod_ref: (mod_ref[i], 0, 0)),
        ],
        out_specs=pl.BlockSpec((1, 1, _NSTEP * 128), lambda i, mod_ref: (i, 0, 0)),
    )
    codes = pl.pallas_call(
        _codes_body,
        grid_spec=codes_spec,
        out_shape=jax.ShapeDtypeStruct((b, 1, _NSTEP * 128), jnp.int32),
    )(mod, input, codebooks)

    chunk = _CN * k
    out = pl.pallas_call(
        _onehot_body,
        grid=(_NSTEP,),
        in_specs=[pl.BlockSpec((b, 1, 128), lambda s: (0, 0, s))],
        out_specs=pl.BlockSpec((b, chunk), lambda s: (0, s)),
        out_shape=jax.ShapeDtypeStruct((b, n * k), jnp.int32),
    )(codes)
    return out


# speedup vs baseline: 2.8554x; 1.1813x over previous
import jax
import jax.numpy as jnp
from jax.experimental import pallas as pl
from jax.experimental.pallas import tpu as pltpu

_CN = 128


def _body(mod_ref, x_ref, cb_ref, out_ref, codes_scr, c2_scr):
    nb = x_ref.shape[0]
    cn = x_ref.shape[1]
    m = cb_ref.shape[0]
    k = cb_ref.shape[1]

    @pl.when(pl.program_id(0) == 0)
    def _init_c2():
        ones = jnp.ones((1, cb_ref.shape[2]), jnp.float32)
        for mm in range(m):
            cbm = cb_ref[mm]
            c2_scr[mm] = jax.lax.dot_general(
                cbm * cbm, ones, (((1,), (1,)), ((), ())),
                preferred_element_type=jnp.float32)

    iota_k = jax.lax.broadcasted_iota(jnp.int32, (k, cn), 0)
    for b in range(nb):
        cb = cb_ref[mod_ref[b]]
        xm2 = x_ref[b] * -2.0
        cross_t = jax.lax.dot_general(
            cb, xm2, (((1,), (1,)), ((), ())),
            preferred_element_type=jnp.float32)
        dists = c2_scr[mod_ref[b]] + cross_t
        minval = jnp.min(dists, axis=0, keepdims=True)
        code = jnp.min(jnp.where(dists == minval, iota_k, k),
                       axis=0, keepdims=True)
        codes_scr[b:b + 1, :] = code

    iota = jax.lax.broadcasted_iota(jnp.int32, (nb, k), 1)
    for j in range(cn):
        cvec = codes_scr[:, j:j + 1]
        out_ref[:, j * k:(j + 1) * k] = (iota == cvec).astype(jnp.int32)


def kernel(input, mod, codebooks):
    b, n, d = input.shape
    m, k, _ = codebooks.shape
    nsteps = pl.cdiv(n, _CN)

    grid_spec = pltpu.PrefetchScalarGridSpec(
        num_scalar_prefetch=1,
        grid=(nsteps,),
        in_specs=[
            pl.BlockSpec((b, _CN, d), lambda s, mod_ref: (0, s, 0)),
            pl.BlockSpec((m, k, d), lambda s, mod_ref: (0, 0, 0)),
        ],
        out_specs=pl.BlockSpec((b, _CN * k), lambda s, mod_ref: (0, s)),
        scratch_shapes=[
            pltpu.VMEM((b, _CN), jnp.int32),
            pltpu.VMEM((m, k, 1), jnp.float32),
        ],
    )
    out = pl.pallas_call(
        _body,
        grid_spec=grid_spec,
        out_shape=jax.ShapeDtypeStruct((b, n * k), jnp.int32),
    )(mod, input, codebooks)
    return out
